# baseline (speedup 1.0000x reference)
import functools

import jax
import jax.numpy as jnp
from jax import lax
from jax.experimental import pallas as pl
from jax.experimental.pallas import tpu as pltpu
from jax.experimental.pallas import tpu_sc as plsc

N = 10000
E = 320000
D = 128
NDEPTH = 2
NPASS = 4
NIN = 5
NSPLIT = 2
E2 = E // NSPLIT

NC = 2
NS = 16
NW = NC * NS
NPAD = 10240
HALF = NPAD // 2
ACCR = HALF + 128
RPS = HALF // NS

_sc_mesh = plsc.VectorSubcoreMesh(core_axis_name="c", subcore_axis_name="s")


def _make_gather(e_n, ch):
    epw = e_n // NW
    nchunk = epw // ch

    @functools.partial(
        pl.kernel,
        out_type=jax.ShapeDtypeStruct((e_n, D), jnp.float32),
        mesh=_sc_mesh,
        scratch_types=[
            pltpu.VMEM((epw,), jnp.int32),
            pltpu.VMEM((epw,), jnp.int32),
            pltpu.VMEM((2, ch, D), jnp.float32),
            pltpu.VMEM((2, ch, D), jnp.float32),
            pltpu.VMEM((2, ch, D), jnp.float32),
            pltpu.SemaphoreType.DMA,
            pltpu.SemaphoreType.DMA,
            pltpu.SemaphoreType.DMA,
            pltpu.SemaphoreType.DMA,
        ],
    )
    def gather(vv, sidx_h, didx_h, out, sidx, didx, sbuf, dbuf, obuf,
               ssem, dsem, osem, isem):
        wid = lax.axis_index("s") * NC + lax.axis_index("c")
        base = wid * epw

        pltpu.async_copy(sidx_h.at[pl.ds(base, epw)], sidx, isem)
        pltpu.async_copy(didx_h.at[pl.ds(base, epw)], didx, isem)
        pltpu.make_async_copy(sidx_h.at[pl.ds(base, epw)], sidx, isem).wait()
        pltpu.make_async_copy(didx_h.at[pl.ds(base, epw)], didx, isem).wait()
        pltpu.async_copy(vv.at[sidx.at[pl.ds(0, ch)]], sbuf.at[0], ssem)
        pltpu.async_copy(vv.at[didx.at[pl.ds(0, ch)]], dbuf.at[0], dsem)

        def body(j, _):
            slot = lax.rem(j, 2)
            nslot = lax.rem(j + 1, 2)

            @pl.when(j + 1 < nchunk)
            def _():
                pltpu.async_copy(vv.at[sidx.at[pl.ds((j + 1) * ch, ch)]],
                                 sbuf.at[nslot], ssem)
                pltpu.async_copy(vv.at[didx.at[pl.ds((j + 1) * ch, ch)]],
                                 dbuf.at[nslot], dsem)

            pltpu.make_async_copy(vv.at[sidx.at[pl.ds(j * ch, ch)]],
                                  sbuf.at[slot], ssem).wait()
            pltpu.make_async_copy(vv.at[didx.at[pl.ds(j * ch, ch)]],
                                  dbuf.at[slot], dsem).wait()

            @pl.when(j >= 2)
            def _():
                pltpu.make_async_copy(
                    obuf.at[slot], out.at[pl.ds(base + (j - 2) * ch, ch)], osem
                ).wait()

            def crow(r, _):
                for c in range(D // 16):
                    sl = pl.ds(c * 16, 16)
                    obuf[slot, r, sl] = sbuf[slot, r, sl] - dbuf[slot, r, sl]
                return 0

            lax.fori_loop(0, ch, crow, 0)
            pltpu.async_copy(obuf.at[slot],
                             out.at[pl.ds(base + j * ch, ch)], osem)
            return 0

        lax.fori_loop(0, nchunk, body, 0)
        for jj in (nchunk - 2, nchunk - 1):
            pltpu.make_async_copy(
                obuf.at[jj % 2], out.at[pl.ds(base + jj * ch, ch)], osem
            ).wait()

    return gather


def _make_scatter(e_n, ch):
    eps = e_n // NS
    nchunk = eps // ch

    @functools.partial(
        pl.kernel,
        out_type=jax.ShapeDtypeStruct((NC, HALF, D), jnp.float32),
        mesh=_sc_mesh,
        scratch_types=[
            pltpu.VMEM((eps,), jnp.int32),
            pltpu.VMEM((nchunk, ch), jnp.int32),
            pltpu.VMEM((2, ch, D), jnp.float32),
            pltpu.VMEM((RPS // 2, D), jnp.float32),
            pltpu.VMEM_SHARED((ACCR, D), jnp.float32),
            pltpu.SemaphoreType.DMA,
            pltpu.SemaphoreType.DMA,
        ],
    )
    def scatter(e_h, didx_h, out, didx1, didx, ebuf, zbuf, acc, esem, isem):
        cid = lax.axis_index("c")
        sid = lax.axis_index("s")
        base = sid * eps

        pltpu.async_copy(didx_h.at[pl.ds(base, eps)], didx1, isem)

        def zrow(r, _):
            for c in range(D // 16):
                zbuf[r, pl.ds(c * 16, 16)] = jnp.zeros((16,), jnp.float32)
            return 0

        lax.fori_loop(0, RPS // 2, zrow, 0)
        for t in range(2):
            pltpu.sync_copy(
                zbuf, acc.at[pl.ds(sid * RPS + t * (RPS // 2), RPS // 2)])

        pltpu.make_async_copy(didx_h.at[pl.ds(base, eps)], didx1, isem).wait()

        off = cid * HALF

        def rrow(r, _):
            for c in range(ch // 16):
                sl = pl.ds(c * 16, 16)
                vi = didx1[pl.ds(r * ch + c * 16, 16)]
                u = vi - off
                ok = (u >= 0) & (u < HALF)
                didx[r, sl] = jnp.where(ok, u, HALF + (vi & 127))
            return 0

        lax.fori_loop(0, nchunk, rrow, 0)
        plsc.subcore_barrier()
        pltpu.async_copy(e_h.at[pl.ds(base, ch)], ebuf.at[0], esem)

        def body(j, _):
            slot = lax.rem(j, 2)
            nslot = lax.rem(j + 1, 2)

            @pl.when(j + 1 < nchunk)
            def _():
                pltpu.async_copy(
                    e_h.at[pl.ds(base + (j + 1) * ch, ch)], ebuf.at[nslot],
                    esem)

            pltpu.make_async_copy(
                e_h.at[pl.ds(base + j * ch, ch)], ebuf.at[slot], esem).wait()
            pltpu.sync_copy(ebuf.at[slot], acc.at[didx.at[j]], add=True)
            return 0

        lax.fori_loop(0, nchunk, body, 0)
        plsc.subcore_barrier()
        pltpu.sync_copy(
            acc.at[pl.ds(sid * RPS, RPS)], out.at[cid, pl.ds(sid * RPS, RPS)]
        )

    return scatter


_gather_c = _make_gather(E2, 40)
_scatter_c = _make_scatter(E2, 80)


def _bdot(a, b):
    return jnp.dot(a.astype(jnp.bfloat16), b.astype(jnp.bfloat16),
                   preferred_element_type=jnp.float32)


def _res_mlp(x, w1, b1, w2, b2, g, bta):
    for i in range(NDEPTH):
        h = jnp.maximum(_bdot(x, w1[i]) + b1[i][None, :], 0.0)
        h = jnp.maximum(_bdot(h, w2[i]) + b2[i][None, :], 0.0)
        mu = jnp.mean(h, axis=-1, keepdims=True)
        d = h - mu
        var = jnp.mean(d * d, axis=-1, keepdims=True)
        h = d * lax.rsqrt(var + 1e-5) * g[i][None, :] + bta[i][None, :]
        x = x + h
    return x


def _wspec(shape):
    return pl.BlockSpec(shape, lambda i: tuple(0 for _ in shape))


_W_SPECS = [
    _wspec((NDEPTH, D, D)), _wspec((NDEPTH, D)),
    _wspec((NDEPTH, D, D)), _wspec((NDEPTH, D)),
    _wspec((NDEPTH, D)), _wspec((NDEPTH, D)),
]

BE = 4000
BN = 2000


def _edge_body0(x_ref, w1, b1, w2, b2, g, bta, out_ref):
    out_ref[...] = _res_mlp(x_ref[...], w1, b1, w2, b2, g, bta)


def _edge_body1(x_ref, e_ref, w1, b1, w2, b2, g, bta, out_ref):
    x = x_ref[...] + e_ref[...]
    out_ref[...] = _res_mlp(x, w1, b1, w2, b2, g, bta)


def _make_edge(e_n):
    call0 = pl.pallas_call(
        _edge_body0,
        grid=(e_n // BE,),
        in_specs=[pl.BlockSpec((BE, D), lambda i: (i, 0))] + _W_SPECS,
        out_specs=pl.BlockSpec((BE, D), lambda i: (i, 0)),
        out_shape=jax.ShapeDtypeStruct((e_n, D), jnp.float32),
    )
    call1 = pl.pallas_call(
        _edge_body1,
        grid=(e_n // BE,),
        in_specs=[pl.BlockSpec((BE, D), lambda i: (i, 0)),
                  pl.BlockSpec((BE, D), lambda i: (i, 0))] + _W_SPECS,
        out_specs=pl.BlockSpec((BE, D), lambda i: (i, 0)),
        out_shape=jax.ShapeDtypeStruct((e_n, D), jnp.float32),
    )
    return call0, call1


_edge_call0, _edge_call1 = _make_edge(E2)


def _node_body(vv_ref, a0_ref, a1_ref, w1, b1, w2, b2, g, bta, out_ref):
    x = vv_ref[...] + a0_ref[...] + a1_ref[...]
    out_ref[...] = _res_mlp(x, w1, b1, w2, b2, g, bta)


_node_call = pl.pallas_call(
    _node_body,
    grid=(N // BN,),
    in_specs=[pl.BlockSpec((BN, D), lambda i: (i, 0)),
              pl.BlockSpec((BN, D), lambda i: (i, 0)),
              pl.BlockSpec((BN, D), lambda i: (i, 0))] + _W_SPECS,
    out_specs=pl.BlockSpec((BN, D), lambda i: (i, 0)),
    out_shape=jax.ShapeDtypeStruct((N, D), jnp.float32),
)


def _enc_body(v_ref, w_ref, b_ref, out_ref):
    out_ref[...] = (
        jnp.dot(v_ref[...], w_ref[...], preferred_element_type=jnp.float32)
        + b_ref[...][None, :]
    )


_enc_call = pl.pallas_call(
    _enc_body,
    grid=(N // BN,),
    in_specs=[pl.BlockSpec((BN, 8), lambda i: (i, 0)),
              _wspec((8, D)), _wspec((D,))],
    out_specs=pl.BlockSpec((BN, D), lambda i: (i, 0)),
    out_shape=jax.ShapeDtypeStruct((N, D), jnp.float32),
)

_dec_call = pl.pallas_call(
    _enc_body,
    grid=(N // BN,),
    in_specs=[pl.BlockSpec((BN, D), lambda i: (i, 0)),
              _wspec((D, 8)), _wspec((8,))],
    out_specs=pl.BlockSpec((BN, 8), lambda i: (i, 0)),
    out_shape=jax.ShapeDtypeStruct((N, 8), jnp.float32),
)


def kernel(v, ij, enc_W, enc_b, dec_W, dec_b,
           edge_W1, edge_b1, edge_W2, edge_b2, edge_g, edge_beta,
           node_W1, node_b1, node_W2, node_b2, node_g, node_beta):
    srcs = [ij[0, c * E2:(c + 1) * E2] for c in range(NSPLIT)]
    dsts = [ij[1, c * E2:(c + 1) * E2] for c in range(NSPLIT)]
    v8 = jnp.pad(v.reshape(N, NIN), ((0, 0), (0, 8 - NIN)))
    encW8 = jnp.pad(enc_W, ((0, 8 - NIN), (0, 0)))
    decW8 = jnp.pad(dec_W, ((0, 0), (0, 8 - NIN)))
    decb8 = jnp.pad(dec_b, (0, 8 - NIN))
    ew = (edge_W1, edge_b1, edge_W2, edge_b2, edge_g, edge_beta)
    nw = (node_W1, node_b1, node_W2, node_b2, node_g, node_beta)

    vv = _enc_call(v8, encW8, enc_b)
    es = [None] * NSPLIT
    for _ in range(NPASS):
        xs = [_gather_c(vv, srcs[c], dsts[c]) for c in range(NSPLIT)]
        es = [
            _edge_call0(xs[c], *ew) if es[c] is None
            else _edge_call1(xs[c], es[c], *ew)
            for c in range(NSPLIT)
        ]
        accs = [_scatter_c(es[c], dsts[c]) for c in range(NSPLIT)]
        vv = _node_call(vv, accs[0].reshape(NPAD, D), accs[1].reshape(NPAD, D),
                        *nw)
    out8 = _dec_call(vv, decW8, decb8)
    return out8[:, :NIN].reshape(1, N, NIN)


# speedup vs baseline: 1.0394x; 1.0394x over previous
import functools

import jax
import jax.numpy as jnp
from jax import lax
from jax.experimental import pallas as pl
from jax.experimental.pallas import tpu as pltpu
from jax.experimental.pallas import tpu_sc as plsc

N = 10000
E = 320000
D = 128
NDEPTH = 2
NPASS = 4
NIN = 5
NSPLIT = 2
E2 = E // NSPLIT

NC = 2
NS = 16
NW = NC * NS
NPAD = 10240
HALF = NPAD // 2
ACCR = HALF + 128
RPS = HALF // NS

_sc_mesh = plsc.VectorSubcoreMesh(core_axis_name="c", subcore_axis_name="s")


def _make_gather(e_n, ch):
    epw = e_n // NW
    nchunk = epw // ch

    @functools.partial(
        pl.kernel,
        out_type=jax.ShapeDtypeStruct((e_n, D), jnp.float32),
        mesh=_sc_mesh,
        scratch_types=[
            pltpu.VMEM((epw,), jnp.int32),
            pltpu.VMEM((epw,), jnp.int32),
            pltpu.VMEM((2, ch, D), jnp.float32),
            pltpu.VMEM((2, ch, D), jnp.float32),
            pltpu.VMEM((2, ch, D), jnp.float32),
            pltpu.SemaphoreType.DMA,
            pltpu.SemaphoreType.DMA,
            pltpu.SemaphoreType.DMA,
            pltpu.SemaphoreType.DMA,
        ],
    )
    def gather(vv, sidx_h, didx_h, out, sidx, didx, sbuf, dbuf, obuf,
               ssem, dsem, osem, isem):
        wid = lax.axis_index("s") * NC + lax.axis_index("c")
        base = wid * epw

        pltpu.async_copy(sidx_h.at[pl.ds(base, epw)], sidx, isem)
        pltpu.async_copy(didx_h.at[pl.ds(base, epw)], didx, isem)
        pltpu.make_async_copy(sidx_h.at[pl.ds(base, epw)], sidx, isem).wait()
        pltpu.make_async_copy(didx_h.at[pl.ds(base, epw)], didx, isem).wait()
        pltpu.async_copy(vv.at[sidx.at[pl.ds(0, ch)]], sbuf.at[0], ssem)
        pltpu.async_copy(vv.at[didx.at[pl.ds(0, ch)]], dbuf.at[0], dsem)

        def body(j, _):
            slot = lax.rem(j, 2)
            nslot = lax.rem(j + 1, 2)

            @pl.when(j + 1 < nchunk)
            def _():
                pltpu.async_copy(vv.at[sidx.at[pl.ds((j + 1) * ch, ch)]],
                                 sbuf.at[nslot], ssem)
                pltpu.async_copy(vv.at[didx.at[pl.ds((j + 1) * ch, ch)]],
                                 dbuf.at[nslot], dsem)

            pltpu.make_async_copy(vv.at[sidx.at[pl.ds(j * ch, ch)]],
                                  sbuf.at[slot], ssem).wait()
            pltpu.make_async_copy(vv.at[didx.at[pl.ds(j * ch, ch)]],
                                  dbuf.at[slot], dsem).wait()

            @pl.when(j >= 2)
            def _():
                pltpu.make_async_copy(
                    obuf.at[slot], out.at[pl.ds(base + (j - 2) * ch, ch)], osem
                ).wait()

            def crow(r, _):
                for c in range(D // 16):
                    sl = pl.ds(c * 16, 16)
                    obuf[slot, r, sl] = sbuf[slot, r, sl] - dbuf[slot, r, sl]
                return 0

            lax.fori_loop(0, ch, crow, 0)
            pltpu.async_copy(obuf.at[slot],
                             out.at[pl.ds(base + j * ch, ch)], osem)
            return 0

        lax.fori_loop(0, nchunk, body, 0)
        for jj in (nchunk - 2, nchunk - 1):
            pltpu.make_async_copy(
                obuf.at[jj % 2], out.at[pl.ds(base + jj * ch, ch)], osem
            ).wait()

    return gather


def _make_scatter(e_n, ch):
    eps = e_n // NS
    nchunk = eps // ch

    @functools.partial(
        pl.kernel,
        out_type=jax.ShapeDtypeStruct((NC, HALF, D), jnp.float32),
        mesh=_sc_mesh,
        scratch_types=[
            pltpu.VMEM((eps,), jnp.int32),
            pltpu.VMEM((nchunk, ch), jnp.int32),
            pltpu.VMEM((3, ch, D), jnp.float32),
            pltpu.VMEM((RPS // 2, D), jnp.float32),
            pltpu.VMEM_SHARED((ACCR, D), jnp.float32),
            pltpu.SemaphoreType.DMA,
            pltpu.SemaphoreType.DMA,
            pltpu.SemaphoreType.DMA,
        ],
    )
    def scatter(e_h, didx_h, out, didx1, didx, ebuf, zbuf, acc, esem, isem,
                asem):
        cid = lax.axis_index("c")
        sid = lax.axis_index("s")
        base = sid * eps

        pltpu.async_copy(didx_h.at[pl.ds(base, eps)], didx1, isem)

        def zrow(r, _):
            for c in range(D // 16):
                zbuf[r, pl.ds(c * 16, 16)] = jnp.zeros((16,), jnp.float32)
            return 0

        lax.fori_loop(0, RPS // 2, zrow, 0)
        for t in range(2):
            pltpu.sync_copy(
                zbuf, acc.at[pl.ds(sid * RPS + t * (RPS // 2), RPS // 2)])

        pltpu.make_async_copy(didx_h.at[pl.ds(base, eps)], didx1, isem).wait()

        off = cid * HALF

        def rrow(r, _):
            for c in range(ch // 16):
                sl = pl.ds(c * 16, 16)
                vi = didx1[pl.ds(r * ch + c * 16, 16)]
                u = vi - off
                ok = (u >= 0) & (u < HALF)
                didx[r, sl] = jnp.where(ok, u, HALF + (vi & 127))
            return 0

        lax.fori_loop(0, nchunk, rrow, 0)
        plsc.subcore_barrier()
        pltpu.async_copy(e_h.at[pl.ds(base, ch)], ebuf.at[0], esem)

        def body(j, _):
            slot = lax.rem(j, 3)
            nslot = lax.rem(j + 1, 3)

            @pl.when(j >= 2)
            def _():
                pltpu.make_async_copy(
                    ebuf.at[nslot], acc.at[didx.at[j - 2]], asem).wait()

            @pl.when(j + 1 < nchunk)
            def _():
                pltpu.async_copy(
                    e_h.at[pl.ds(base + (j + 1) * ch, ch)], ebuf.at[nslot],
                    esem)

            pltpu.make_async_copy(
                e_h.at[pl.ds(base + j * ch, ch)], ebuf.at[slot], esem).wait()
            pltpu.async_copy(ebuf.at[slot], acc.at[didx.at[j]], asem,
                             add=True)
            return 0

        lax.fori_loop(0, nchunk, body, 0)
        for jj in (nchunk - 2, nchunk - 1):
            pltpu.make_async_copy(
                ebuf.at[jj % 3], acc.at[didx.at[jj]], asem).wait()
        plsc.subcore_barrier()
        pltpu.sync_copy(
            acc.at[pl.ds(sid * RPS, RPS)], out.at[cid, pl.ds(sid * RPS, RPS)]
        )

    return scatter


_gather_c = _make_gather(E2, 40)
_scatter_c = _make_scatter(E2, 80)


def _bdot(a, b):
    return jnp.dot(a.astype(jnp.bfloat16), b.astype(jnp.bfloat16),
                   preferred_element_type=jnp.float32)


def _res_mlp(x, w1, b1, w2, b2, g, bta):
    for i in range(NDEPTH):
        h = jnp.maximum(_bdot(x, w1[i]) + b1[i][None, :], 0.0)
        h = jnp.maximum(_bdot(h, w2[i]) + b2[i][None, :], 0.0)
        mu = jnp.mean(h, axis=-1, keepdims=True)
        d = h - mu
        var = jnp.mean(d * d, axis=-1, keepdims=True)
        h = d * lax.rsqrt(var + 1e-5) * g[i][None, :] + bta[i][None, :]
        x = x + h
    return x


def _wspec(shape):
    return pl.BlockSpec(shape, lambda i: tuple(0 for _ in shape))


_W_SPECS = [
    _wspec((NDEPTH, D, D)), _wspec((NDEPTH, D)),
    _wspec((NDEPTH, D, D)), _wspec((NDEPTH, D)),
    _wspec((NDEPTH, D)), _wspec((NDEPTH, D)),
]

BE = 4000
BN = 2000


def _edge_body0(x_ref, w1, b1, w2, b2, g, bta, out_ref):
    out_ref[...] = _res_mlp(x_ref[...], w1, b1, w2, b2, g, bta)


def _edge_body1(x_ref, e_ref, w1, b1, w2, b2, g, bta, out_ref):
    x = x_ref[...] + e_ref[...]
    out_ref[...] = _res_mlp(x, w1, b1, w2, b2, g, bta)


def _make_edge(e_n):
    call0 = pl.pallas_call(
        _edge_body0,
        grid=(e_n // BE,),
        in_specs=[pl.BlockSpec((BE, D), lambda i: (i, 0))] + _W_SPECS,
        out_specs=pl.BlockSpec((BE, D), lambda i: (i, 0)),
        out_shape=jax.ShapeDtypeStruct((e_n, D), jnp.float32),
    )
    call1 = pl.pallas_call(
        _edge_body1,
        grid=(e_n // BE,),
        in_specs=[pl.BlockSpec((BE, D), lambda i: (i, 0)),
                  pl.BlockSpec((BE, D), lambda i: (i, 0))] + _W_SPECS,
        out_specs=pl.BlockSpec((BE, D), lambda i: (i, 0)),
        out_shape=jax.ShapeDtypeStruct((e_n, D), jnp.float32),
    )
    return call0, call1


_edge_call0, _edge_call1 = _make_edge(E2)


def _node_body(vv_ref, a0_ref, a1_ref, w1, b1, w2, b2, g, bta, out_ref):
    x = vv_ref[...] + a0_ref[...] + a1_ref[...]
    out_ref[...] = _res_mlp(x, w1, b1, w2, b2, g, bta)


_node_call = pl.pallas_call(
    _node_body,
    grid=(N // BN,),
    in_specs=[pl.BlockSpec((BN, D), lambda i: (i, 0)),
              pl.BlockSpec((BN, D), lambda i: (i, 0)),
              pl.BlockSpec((BN, D), lambda i: (i, 0))] + _W_SPECS,
    out_specs=pl.BlockSpec((BN, D), lambda i: (i, 0)),
    out_shape=jax.ShapeDtypeStruct((N, D), jnp.float32),
)


def _enc_body(v_ref, w_ref, b_ref, out_ref):
    out_ref[...] = (
        jnp.dot(v_ref[...], w_ref[...], preferred_element_type=jnp.float32)
        + b_ref[...][None, :]
    )


_enc_call = pl.pallas_call(
    _enc_body,
    grid=(N // BN,),
    in_specs=[pl.BlockSpec((BN, 8), lambda i: (i, 0)),
              _wspec((8, D)), _wspec((D,))],
    out_specs=pl.BlockSpec((BN, D), lambda i: (i, 0)),
    out_shape=jax.ShapeDtypeStruct((N, D), jnp.float32),
)

_dec_call = pl.pallas_call(
    _enc_body,
    grid=(N // BN,),
    in_specs=[pl.BlockSpec((BN, D), lambda i: (i, 0)),
              _wspec((D, 8)), _wspec((8,))],
    out_specs=pl.BlockSpec((BN, 8), lambda i: (i, 0)),
    out_shape=jax.ShapeDtypeStruct((N, 8), jnp.float32),
)


def kernel(v, ij, enc_W, enc_b, dec_W, dec_b,
           edge_W1, edge_b1, edge_W2, edge_b2, edge_g, edge_beta,
           node_W1, node_b1, node_W2, node_b2, node_g, node_beta):
    srcs = [ij[0, c * E2:(c + 1) * E2] for c in range(NSPLIT)]
    dsts = [ij[1, c * E2:(c + 1) * E2] for c in range(NSPLIT)]
    v8 = jnp.pad(v.reshape(N, NIN), ((0, 0), (0, 8 - NIN)))
    encW8 = jnp.pad(enc_W, ((0, 8 - NIN), (0, 0)))
    decW8 = jnp.pad(dec_W, ((0, 0), (0, 8 - NIN)))
    decb8 = jnp.pad(dec_b, (0, 8 - NIN))
    ew = (edge_W1, edge_b1, edge_W2, edge_b2, edge_g, edge_beta)
    nw = (node_W1, node_b1, node_W2, node_b2, node_g, node_beta)

    vv = _enc_call(v8, encW8, enc_b)
    es = [None] * NSPLIT
    for _ in range(NPASS):
        xs = [_gather_c(vv, srcs[c], dsts[c]) for c in range(NSPLIT)]
        es = [
            _edge_call0(xs[c], *ew) if es[c] is None
            else _edge_call1(xs[c], es[c], *ew)
            for c in range(NSPLIT)
        ]
        accs = [_scatter_c(es[c], dsts[c]) for c in range(NSPLIT)]
        vv = _node_call(vv, accs[0].reshape(NPAD, D), accs[1].reshape(NPAD, D),
                        *nw)
    out8 = _dec_call(vv, decW8, decb8)
    return out8[:, :NIN].reshape(1, N, NIN)


# speedup vs baseline: 1.0470x; 1.0073x over previous
import functools

import jax
import jax.numpy as jnp
from jax import lax
from jax.experimental import pallas as pl
from jax.experimental.pallas import tpu as pltpu
from jax.experimental.pallas import tpu_sc as plsc

N = 10000
E = 320000
D = 128
NDEPTH = 2
NPASS = 4
NIN = 5
NSPLIT = 2
E2 = E // NSPLIT

NC = 2
NS = 16
NW = NC * NS
NPAD = 10240
HALF = NPAD // 2
ACCR = HALF + 128
RPS = HALF // NS

_sc_mesh = plsc.VectorSubcoreMesh(core_axis_name="c", subcore_axis_name="s")


def _make_gather(e_n, ch):
    epw = e_n // NW
    nchunk = epw // ch

    @functools.partial(
        pl.kernel,
        out_type=jax.ShapeDtypeStruct((e_n, D), jnp.float32),
        mesh=_sc_mesh,
        scratch_types=[
            pltpu.VMEM((epw,), jnp.int32),
            pltpu.VMEM((epw,), jnp.int32),
            pltpu.VMEM((3, ch, D), jnp.float32),
            pltpu.VMEM((3, ch, D), jnp.float32),
            pltpu.VMEM((3, ch, D), jnp.float32),
            pltpu.SemaphoreType.DMA,
            pltpu.SemaphoreType.DMA,
            pltpu.SemaphoreType.DMA,
            pltpu.SemaphoreType.DMA,
        ],
    )
    def gather(vv, sidx_h, didx_h, out, sidx, didx, sbuf, dbuf, obuf,
               ssem, dsem, osem, isem):
        wid = lax.axis_index("s") * NC + lax.axis_index("c")
        base = wid * epw

        pltpu.async_copy(sidx_h.at[pl.ds(base, epw)], sidx, isem)
        pltpu.async_copy(didx_h.at[pl.ds(base, epw)], didx, isem)
        pltpu.make_async_copy(sidx_h.at[pl.ds(base, epw)], sidx, isem).wait()
        pltpu.make_async_copy(didx_h.at[pl.ds(base, epw)], didx, isem).wait()
        for p in (0, 1):
            pltpu.async_copy(vv.at[sidx.at[pl.ds(p * ch, ch)]], sbuf.at[p],
                             ssem)
            pltpu.async_copy(vv.at[didx.at[pl.ds(p * ch, ch)]], dbuf.at[p],
                             dsem)

        def body(j, _):
            slot = lax.rem(j, 3)
            nslot = lax.rem(j + 2, 3)

            @pl.when(j + 2 < nchunk)
            def _():
                pltpu.async_copy(vv.at[sidx.at[pl.ds((j + 2) * ch, ch)]],
                                 sbuf.at[nslot], ssem)
                pltpu.async_copy(vv.at[didx.at[pl.ds((j + 2) * ch, ch)]],
                                 dbuf.at[nslot], dsem)

            pltpu.make_async_copy(vv.at[sidx.at[pl.ds(j * ch, ch)]],
                                  sbuf.at[slot], ssem).wait()
            pltpu.make_async_copy(vv.at[didx.at[pl.ds(j * ch, ch)]],
                                  dbuf.at[slot], dsem).wait()

            @pl.when(j >= 3)
            def _():
                pltpu.make_async_copy(
                    obuf.at[slot], out.at[pl.ds(base + (j - 3) * ch, ch)], osem
                ).wait()

            def crow(r, _):
                for c in range(D // 16):
                    sl = pl.ds(c * 16, 16)
                    obuf[slot, r, sl] = sbuf[slot, r, sl] - dbuf[slot, r, sl]
                return 0

            lax.fori_loop(0, ch, crow, 0)
            pltpu.async_copy(obuf.at[slot],
                             out.at[pl.ds(base + j * ch, ch)], osem)
            return 0

        lax.fori_loop(0, nchunk, body, 0)
        for jj in (nchunk - 3, nchunk - 2, nchunk - 1):
            pltpu.make_async_copy(
                obuf.at[jj % 3], out.at[pl.ds(base + jj * ch, ch)], osem
            ).wait()

    return gather


def _make_scatter(e_n, ch):
    eps = e_n // NS
    nchunk = eps // ch

    @functools.partial(
        pl.kernel,
        out_type=jax.ShapeDtypeStruct((NC, HALF, D), jnp.float32),
        mesh=_sc_mesh,
        scratch_types=[
            pltpu.VMEM((eps,), jnp.int32),
            pltpu.VMEM((nchunk, ch), jnp.int32),
            pltpu.VMEM((3, ch, D), jnp.float32),
            pltpu.VMEM((RPS // 2, D), jnp.float32),
            pltpu.VMEM_SHARED((ACCR, D), jnp.float32),
            pltpu.SemaphoreType.DMA,
            pltpu.SemaphoreType.DMA,
            pltpu.SemaphoreType.DMA,
        ],
    )
    def scatter(e_h, didx_h, out, didx1, didx, ebuf, zbuf, acc, esem, isem,
                asem):
        cid = lax.axis_index("c")
        sid = lax.axis_index("s")
        base = sid * eps

        pltpu.async_copy(didx_h.at[pl.ds(base, eps)], didx1, isem)

        def zrow(r, _):
            for c in range(D // 16):
                zbuf[r, pl.ds(c * 16, 16)] = jnp.zeros((16,), jnp.float32)
            return 0

        lax.fori_loop(0, RPS // 2, zrow, 0)
        for t in range(2):
            pltpu.sync_copy(
                zbuf, acc.at[pl.ds(sid * RPS + t * (RPS // 2), RPS // 2)])

        pltpu.make_async_copy(didx_h.at[pl.ds(base, eps)], didx1, isem).wait()

        off = cid * HALF

        def rrow(r, _):
            for c in range(ch // 16):
                sl = pl.ds(c * 16, 16)
                vi = didx1[pl.ds(r * ch + c * 16, 16)]
                u = vi - off
                ok = (u >= 0) & (u < HALF)
                didx[r, sl] = jnp.where(ok, u, HALF + (vi & 127))
            return 0

        lax.fori_loop(0, nchunk, rrow, 0)
        plsc.subcore_barrier()
        pltpu.async_copy(e_h.at[pl.ds(base, ch)], ebuf.at[0], esem)

        def body(j, _):
            slot = lax.rem(j, 3)
            nslot = lax.rem(j + 1, 3)

            @pl.when(j >= 2)
            def _():
                pltpu.make_async_copy(
                    ebuf.at[nslot], acc.at[didx.at[j - 2]], asem).wait()

            @pl.when(j + 1 < nchunk)
            def _():
                pltpu.async_copy(
                    e_h.at[pl.ds(base + (j + 1) * ch, ch)], ebuf.at[nslot],
                    esem)

            pltpu.make_async_copy(
                e_h.at[pl.ds(base + j * ch, ch)], ebuf.at[slot], esem).wait()
            pltpu.async_copy(ebuf.at[slot], acc.at[didx.at[j]], asem,
                             add=True)
            return 0

        lax.fori_loop(0, nchunk, body, 0)
        for jj in (nchunk - 2, nchunk - 1):
            pltpu.make_async_copy(
                ebuf.at[jj % 3], acc.at[didx.at[jj]], asem).wait()
        plsc.subcore_barrier()
        pltpu.sync_copy(
            acc.at[pl.ds(sid * RPS, RPS)], out.at[cid, pl.ds(sid * RPS, RPS)]
        )

    return scatter


_gather_c = _make_gather(E2, 40)
_scatter_c = _make_scatter(E2, 80)


def _bdot(a, b):
    return jnp.dot(a.astype(jnp.bfloat16), b.astype(jnp.bfloat16),
                   preferred_element_type=jnp.float32)


def _res_mlp(x, w1, b1, w2, b2, g, bta):
    for i in range(NDEPTH):
        h = jnp.maximum(_bdot(x, w1[i]) + b1[i][None, :], 0.0)
        h = jnp.maximum(_bdot(h, w2[i]) + b2[i][None, :], 0.0)
        mu = jnp.mean(h, axis=-1, keepdims=True)
        d = h - mu
        var = jnp.mean(d * d, axis=-1, keepdims=True)
        h = d * lax.rsqrt(var + 1e-5) * g[i][None, :] + bta[i][None, :]
        x = x + h
    return x


def _wspec(shape):
    return pl.BlockSpec(shape, lambda i: tuple(0 for _ in shape))


_W_SPECS = [
    _wspec((NDEPTH, D, D)), _wspec((NDEPTH, D)),
    _wspec((NDEPTH, D, D)), _wspec((NDEPTH, D)),
    _wspec((NDEPTH, D)), _wspec((NDEPTH, D)),
]

BE = 4000
BN = 2000


def _edge_body0(x_ref, w1, b1, w2, b2, g, bta, out_ref):
    out_ref[...] = _res_mlp(x_ref[...], w1, b1, w2, b2, g, bta)


def _edge_body1(x_ref, e_ref, w1, b1, w2, b2, g, bta, out_ref):
    x = x_ref[...] + e_ref[...]
    out_ref[...] = _res_mlp(x, w1, b1, w2, b2, g, bta)


def _make_edge(e_n):
    call0 = pl.pallas_call(
        _edge_body0,
        grid=(e_n // BE,),
        in_specs=[pl.BlockSpec((BE, D), lambda i: (i, 0))] + _W_SPECS,
        out_specs=pl.BlockSpec((BE, D), lambda i: (i, 0)),
        out_shape=jax.ShapeDtypeStruct((e_n, D), jnp.float32),
    )
    call1 = pl.pallas_call(
        _edge_body1,
        grid=(e_n // BE,),
        in_specs=[pl.BlockSpec((BE, D), lambda i: (i, 0)),
                  pl.BlockSpec((BE, D), lambda i: (i, 0))] + _W_SPECS,
        out_specs=pl.BlockSpec((BE, D), lambda i: (i, 0)),
        out_shape=jax.ShapeDtypeStruct((e_n, D), jnp.float32),
    )
    return call0, call1


_edge_call0, _edge_call1 = _make_edge(E2)


def _node_body(vv_ref, a0_ref, a1_ref, w1, b1, w2, b2, g, bta, out_ref):
    x = vv_ref[...] + a0_ref[...] + a1_ref[...]
    out_ref[...] = _res_mlp(x, w1, b1, w2, b2, g, bta)


_node_call = pl.pallas_call(
    _node_body,
    grid=(N // BN,),
    in_specs=[pl.BlockSpec((BN, D), lambda i: (i, 0)),
              pl.BlockSpec((BN, D), lambda i: (i, 0)),
              pl.BlockSpec((BN, D), lambda i: (i, 0))] + _W_SPECS,
    out_specs=pl.BlockSpec((BN, D), lambda i: (i, 0)),
    out_shape=jax.ShapeDtypeStruct((N, D), jnp.float32),
)


def _enc_body(v_ref, w_ref, b_ref, out_ref):
    out_ref[...] = (
        jnp.dot(v_ref[...], w_ref[...], preferred_element_type=jnp.float32)
        + b_ref[...][None, :]
    )


_enc_call = pl.pallas_call(
    _enc_body,
    grid=(N // BN,),
    in_specs=[pl.BlockSpec((BN, 8), lambda i: (i, 0)),
              _wspec((8, D)), _wspec((D,))],
    out_specs=pl.BlockSpec((BN, D), lambda i: (i, 0)),
    out_shape=jax.ShapeDtypeStruct((N, D), jnp.float32),
)

_dec_call = pl.pallas_call(
    _enc_body,
    grid=(N // BN,),
    in_specs=[pl.BlockSpec((BN, D), lambda i: (i, 0)),
              _wspec((D, 8)), _wspec((8,))],
    out_specs=pl.BlockSpec((BN, 8), lambda i: (i, 0)),
    out_shape=jax.ShapeDtypeStruct((N, 8), jnp.float32),
)


def kernel(v, ij, enc_W, enc_b, dec_W, dec_b,
           edge_W1, edge_b1, edge_W2, edge_b2, edge_g, edge_beta,
           node_W1, node_b1, node_W2, node_b2, node_g, node_beta):
    srcs = [ij[0, c * E2:(c + 1) * E2] for c in range(NSPLIT)]
    dsts = [ij[1, c * E2:(c + 1) * E2] for c in range(NSPLIT)]
    v8 = jnp.pad(v.reshape(N, NIN), ((0, 0), (0, 8 - NIN)))
    encW8 = jnp.pad(enc_W, ((0, 8 - NIN), (0, 0)))
    decW8 = jnp.pad(dec_W, ((0, 0), (0, 8 - NIN)))
    decb8 = jnp.pad(dec_b, (0, 8 - NIN))
    ew = (edge_W1, edge_b1, edge_W2, edge_b2, edge_g, edge_beta)
    nw = (node_W1, node_b1, node_W2, node_b2, node_g, node_beta)

    vv = _enc_call(v8, encW8, enc_b)
    es = [None] * NSPLIT
    for _ in range(NPASS):
        xs = [_gather_c(vv, srcs[c], dsts[c]) for c in range(NSPLIT)]
        es = [
            _edge_call0(xs[c], *ew) if es[c] is None
            else _edge_call1(xs[c], es[c], *ew)
            for c in range(NSPLIT)
        ]
        accs = [_scatter_c(es[c], dsts[c]) for c in range(NSPLIT)]
        vv = _node_call(vv, accs[0].reshape(NPAD, D), accs[1].reshape(NPAD, D),
                        *nw)
    out8 = _dec_call(vv, decW8, decb8)
    return out8[:, :NIN].reshape(1, N, NIN)
